# baseline (device time: 167357 ns/iter reference)
import jax
import jax.numpy as jnp
from jax import lax
from jax.experimental import pallas as pl
from jax.experimental.pallas import tpu as pltpu

N_DEV = 4


def kernel(dy, W):
    m, k = dy.shape
    d, _ = W.shape

    def body(dy_ref, w_ref, out_ref, comm_ref, send_sems, recv_sems):
        my_pos = lax.axis_index("i")
        left = (my_pos - 1) % N_DEV
        right = (my_pos + 1) % N_DEV

        barrier_sem = pltpu.get_barrier_semaphore()
        for nbr in [left, right]:
            pl.semaphore_signal(
                barrier_sem, inc=1,
                device_id=(nbr,), device_id_type=pl.DeviceIdType.MESH,
            )
        pl.semaphore_wait(barrier_sem, 2)

        partial = lax.dot_general(
            dy_ref[...], w_ref[...],
            dimension_numbers=(((1,), (1,)), ((), ())),
            preferred_element_type=jnp.float32,
        )
        out_ref[...] = partial
        comm_ref[0] = partial

        for h in range(N_DEV - 1):
            send_slot = h % 2
            recv_slot = (h + 1) % 2
            rdma = pltpu.make_async_remote_copy(
                src_ref=comm_ref.at[send_slot],
                dst_ref=comm_ref.at[recv_slot],
                send_sem=send_sems.at[send_slot],
                recv_sem=recv_sems.at[recv_slot],
                device_id=(right,),
                device_id_type=pl.DeviceIdType.MESH,
            )
            rdma.start()
            rdma.wait()
            out_ref[...] += comm_ref[recv_slot]

    return pl.pallas_call(
        body,
        out_shape=jax.ShapeDtypeStruct((m, d), jnp.float32),
        in_specs=[
            pl.BlockSpec(memory_space=pltpu.VMEM),
            pl.BlockSpec(memory_space=pltpu.VMEM),
        ],
        out_specs=pl.BlockSpec(memory_space=pltpu.VMEM),
        scratch_shapes=[
            pltpu.VMEM((2, m, d), jnp.float32),
            pltpu.SemaphoreType.DMA((2,)),
            pltpu.SemaphoreType.DMA((2,)),
        ],
        compiler_params=pltpu.CompilerParams(collective_id=0),
    )(dy, W)


# device time: 66606 ns/iter; 2.5126x vs baseline; 2.5126x over previous
import jax
import jax.numpy as jnp
from jax import lax
from jax.experimental import pallas as pl
from jax.experimental.pallas import tpu as pltpu

N_DEV = 4


def kernel(dy, W):
    m, k = dy.shape
    d, _ = W.shape
    mh = m // 2
    mq = m // 4
    dh = d // 2

    def body(dy_ref, w_ref, out_ref, c1a, c1b, c2a, c2b, send_sems, recv_sems):
        p = lax.axis_index("i")
        x = p // 2
        y = (p // 2) ^ (p % 2)
        px = 3 - p
        py = p ^ 1

        barrier_sem = pltpu.get_barrier_semaphore()
        for nbr in [px, py]:
            pl.semaphore_signal(
                barrier_sem, inc=1,
                device_id=(nbr,), device_id_type=pl.DeviceIdType.MESH,
            )
        pl.semaphore_wait(barrier_sem, 2)

        out_ref[...] = lax.dot_general(
            dy_ref[...], w_ref[...],
            dimension_numbers=(((1,), (1,)), ((), ())),
            preferred_element_type=jnp.float32,
        )

        cols_a = pl.ds(0, dh)
        cols_b = pl.ds(dh, dh)

        def xchg(idx, src, dst, partner):
            rdma = pltpu.make_async_remote_copy(
                src_ref=src, dst_ref=dst,
                send_sem=send_sems.at[idx], recv_sem=recv_sems.at[idx],
                device_id=(partner,), device_id_type=pl.DeviceIdType.MESH,
            )
            rdma.start()
            return rdma

        a1 = xchg(0, out_ref.at[pl.ds((1 - x) * mh, mh), cols_a], c1a, px)
        b1 = xchg(1, out_ref.at[pl.ds((1 - y) * mh, mh), cols_b], c1b, py)
        a1.wait()
        b1.wait()
        out_ref[pl.ds(x * mh, mh), cols_a] += c1a[...]
        out_ref[pl.ds(y * mh, mh), cols_b] += c1b[...]

        ra = x * mh + y * mq
        rb = y * mh + x * mq
        a2 = xchg(2, out_ref.at[pl.ds(x * mh + (1 - y) * mq, mq), cols_a], c2a, py)
        b2 = xchg(3, out_ref.at[pl.ds(y * mh + (1 - x) * mq, mq), cols_b], c2b, px)
        a2.wait()
        b2.wait()
        out_ref[pl.ds(ra, mq), cols_a] += c2a[...]
        out_ref[pl.ds(rb, mq), cols_b] += c2b[...]

        a3 = xchg(4, out_ref.at[pl.ds(ra, mq), cols_a],
                  out_ref.at[pl.ds(ra, mq), cols_a], py)
        b3 = xchg(5, out_ref.at[pl.ds(rb, mq), cols_b],
                  out_ref.at[pl.ds(rb, mq), cols_b], px)
        a3.wait()
        b3.wait()

        a4 = xchg(6, out_ref.at[pl.ds(x * mh, mh), cols_a],
                  out_ref.at[pl.ds(x * mh, mh), cols_a], px)
        b4 = xchg(7, out_ref.at[pl.ds(y * mh, mh), cols_b],
                  out_ref.at[pl.ds(y * mh, mh), cols_b], py)
        a4.wait()
        b4.wait()

    return pl.pallas_call(
        body,
        out_shape=jax.ShapeDtypeStruct((m, d), jnp.float32),
        in_specs=[
            pl.BlockSpec(memory_space=pltpu.VMEM),
            pl.BlockSpec(memory_space=pltpu.VMEM),
        ],
        out_specs=pl.BlockSpec(memory_space=pltpu.VMEM),
        scratch_shapes=[
            pltpu.VMEM((mh, dh), jnp.float32),
            pltpu.VMEM((mh, dh), jnp.float32),
            pltpu.VMEM((mq, dh), jnp.float32),
            pltpu.VMEM((mq, dh), jnp.float32),
            pltpu.SemaphoreType.DMA((8,)),
            pltpu.SemaphoreType.DMA((8,)),
        ],
        compiler_params=pltpu.CompilerParams(collective_id=0),
    )(dy, W)


# device time: 48825 ns/iter; 3.4277x vs baseline; 1.3642x over previous
import jax
import jax.numpy as jnp
from jax import lax
from jax.experimental import pallas as pl
from jax.experimental.pallas import tpu as pltpu

N_DEV = 4
BF16 = jnp.bfloat16


def kernel(dy, W):
    m, k = dy.shape
    d, _ = W.shape
    mh = m // 2
    mq = m // 4
    dh = d // 2

    def body(dy_ref, w_ref, out_ref,
             dyb, wb,
             s1a, s1b, c1a, c1b,
             s2a, s2b, c2a, c2b,
             s3a, s3b, r3a, r3b,
             r4a, r4b,
             send_sems, recv_sems):
        p = lax.axis_index("i")
        x = p // 2
        y = (p // 2) ^ (p % 2)
        px = 3 - p
        py = p ^ 1

        barrier_sem = pltpu.get_barrier_semaphore()
        for nbr in [px, py]:
            pl.semaphore_signal(
                barrier_sem, inc=1,
                device_id=(nbr,), device_id_type=pl.DeviceIdType.MESH,
            )
        pl.semaphore_wait(barrier_sem, 2)

        cols_a = pl.ds(0, dh)
        cols_b = pl.ds(dh, dh)

        def mk(idx, src, dst, partner):
            return pltpu.make_async_remote_copy(
                src_ref=src, dst_ref=dst,
                send_sem=send_sems.at[idx], recv_sem=recv_sems.at[idx],
                device_id=(partner,), device_id_type=pl.DeviceIdType.MESH,
            )

        def block_dot(rows, wcol0):
            return lax.dot_general(
                dyb[rows, :], wb[pl.ds(wcol0, dh), :],
                dimension_numbers=(((1,), (1,)), ((), ())),
                preferred_element_type=jnp.float32,
            )

        dyb[...] = dy_ref[...].astype(BF16)
        wb[...] = w_ref[...].astype(BF16)

        s1a[...] = block_dot(pl.ds((1 - x) * mh, mh), 0).astype(BF16)
        a1 = mk(0, s1a, c1a, px)
        a1.start()
        s1b[...] = block_dot(pl.ds((1 - y) * mh, mh), dh).astype(BF16)
        b1 = mk(1, s1b, c1b, py)
        b1.start()

        out_ref[pl.ds(x * mh, mh), cols_a] = block_dot(pl.ds(x * mh, mh), 0)
        out_ref[pl.ds(y * mh, mh), cols_b] = block_dot(pl.ds(y * mh, mh), dh)

        ra = x * mh + y * mq
        qa = x * mh + (1 - y) * mq
        rb = y * mh + x * mq
        qb = y * mh + (1 - x) * mq

        a1.wait_recv()
        out_ref[pl.ds(qa, mq), cols_a] += c1a[pl.ds((1 - y) * mq, mq), :].astype(jnp.float32)
        s2a[...] = out_ref[pl.ds(qa, mq), cols_a].astype(BF16)
        a2 = mk(2, s2a, c2a, py)
        a2.start()
        out_ref[pl.ds(ra, mq), cols_a] += c1a[pl.ds(y * mq, mq), :].astype(jnp.float32)

        b1.wait_recv()
        out_ref[pl.ds(qb, mq), cols_b] += c1b[pl.ds((1 - x) * mq, mq), :].astype(jnp.float32)
        s2b[...] = out_ref[pl.ds(qb, mq), cols_b].astype(BF16)
        b2 = mk(3, s2b, c2b, px)
        b2.start()
        out_ref[pl.ds(rb, mq), cols_b] += c1b[pl.ds(x * mq, mq), :].astype(jnp.float32)

        a2.wait_recv()
        out_ref[pl.ds(ra, mq), cols_a] += c2a[...].astype(jnp.float32)
        s3a[...] = out_ref[pl.ds(ra, mq), cols_a].astype(BF16)
        a3 = mk(4, s3a, r3a, py)
        a3.start()
        a4i = mk(6, s3a, r4a.at[pl.ds(y * mq, mq), :], px)
        a4i.start()

        b2.wait_recv()
        out_ref[pl.ds(rb, mq), cols_b] += c2b[...].astype(jnp.float32)
        s3b[...] = out_ref[pl.ds(rb, mq), cols_b].astype(BF16)
        b3 = mk(5, s3b, r3b, px)
        b3.start()
        b4i = mk(7, s3b, r4b.at[pl.ds(x * mq, mq), :], py)
        b4i.start()

        a3.wait_recv()
        a4f = mk(8, r3a, r4a.at[pl.ds((1 - y) * mq, mq), :], px)
        a4f.start()
        out_ref[pl.ds(qa, mq), cols_a] = r3a[...].astype(jnp.float32)

        b3.wait_recv()
        b4f = mk(9, r3b, r4b.at[pl.ds((1 - x) * mq, mq), :], py)
        b4f.start()
        out_ref[pl.ds(qb, mq), cols_b] = r3b[...].astype(jnp.float32)

        a4i.wait_recv()
        a4f.wait_recv()
        out_ref[pl.ds((1 - x) * mh, mh), cols_a] = r4a[...].astype(jnp.float32)
        b4i.wait_recv()
        b4f.wait_recv()
        out_ref[pl.ds((1 - y) * mh, mh), cols_b] = r4b[...].astype(jnp.float32)

        for r in [a1, b1, a2, b2, a3, b3, a4i, b4i, a4f, b4f]:
            r.wait_send()

    return pl.pallas_call(
        body,
        out_shape=jax.ShapeDtypeStruct((m, d), jnp.float32),
        in_specs=[
            pl.BlockSpec(memory_space=pltpu.VMEM),
            pl.BlockSpec(memory_space=pltpu.VMEM),
        ],
        out_specs=pl.BlockSpec(memory_space=pltpu.VMEM),
        scratch_shapes=[
            pltpu.VMEM((m, k), BF16),
            pltpu.VMEM((d, k), BF16),
            pltpu.VMEM((mh, dh), BF16),
            pltpu.VMEM((mh, dh), BF16),
            pltpu.VMEM((mh, dh), BF16),
            pltpu.VMEM((mh, dh), BF16),
            pltpu.VMEM((mq, dh), BF16),
            pltpu.VMEM((mq, dh), BF16),
            pltpu.VMEM((mq, dh), BF16),
            pltpu.VMEM((mq, dh), BF16),
            pltpu.VMEM((mq, dh), BF16),
            pltpu.VMEM((mq, dh), BF16),
            pltpu.VMEM((mq, dh), BF16),
            pltpu.VMEM((mq, dh), BF16),
            pltpu.VMEM((mh, dh), BF16),
            pltpu.VMEM((mh, dh), BF16),
            pltpu.SemaphoreType.DMA((10,)),
            pltpu.SemaphoreType.DMA((10,)),
        ],
        compiler_params=pltpu.CompilerParams(
            collective_id=0, vmem_limit_bytes=100 * 1024 * 1024,
        ),
    )(dy, W)


# device time: 47086 ns/iter; 3.5543x vs baseline; 1.0369x over previous
import jax
import jax.numpy as jnp
from jax import lax
from jax.experimental import pallas as pl
from jax.experimental.pallas import tpu as pltpu

N_DEV = 4
BF16 = jnp.bfloat16
F32 = jnp.float32


def kernel(dy, W):
    m, k = dy.shape
    d, _ = W.shape
    mh = m // 2
    mq = m // 4
    dh = d // 2

    def body(dy_ref, w_ref, out_ref,
             s1a, s1b, c1a, c1b,
             s2a, s2b, c2a, c2b,
             s3a, s3b, r3a, r3b,
             r4a, r4b,
             send_sems, recv_sems):
        p = lax.axis_index("i")
        x = p // 2
        y = (p // 2) ^ (p % 2)
        px = 3 - p
        py = p ^ 1

        barrier_sem = pltpu.get_barrier_semaphore()
        for nbr in [px, py]:
            pl.semaphore_signal(
                barrier_sem, inc=1,
                device_id=(nbr,), device_id_type=pl.DeviceIdType.MESH,
            )
        pl.semaphore_wait(barrier_sem, 2)

        cols_a = pl.ds(0, dh)
        cols_b = pl.ds(dh, dh)

        def mk(idx, src, dst, partner):
            return pltpu.make_async_remote_copy(
                src_ref=src, dst_ref=dst,
                send_sem=send_sems.at[idx], recv_sem=recv_sems.at[idx],
                device_id=(partner,), device_id_type=pl.DeviceIdType.MESH,
            )

        def qdot(row0, wcol0):
            return lax.dot_general(
                dy_ref[pl.ds(row0, mq), :], w_ref[pl.ds(wcol0, dh), :],
                dimension_numbers=(((1,), (1,)), ((), ())),
                preferred_element_type=F32,
            )

        ra = x * mh + y * mq
        qa = x * mh + (1 - y) * mq
        rb = y * mh + x * mq
        qb = y * mh + (1 - x) * mq

        s1a[pl.ds((1 - y) * mq, mq), :] = qdot((1 - x) * mh + (1 - y) * mq, 0).astype(BF16)
        a1a = mk(0, s1a.at[pl.ds((1 - y) * mq, mq), :],
                 c1a.at[pl.ds((1 - y) * mq, mq), :], px)
        a1a.start()
        s1b[pl.ds((1 - x) * mq, mq), :] = qdot((1 - y) * mh + (1 - x) * mq, dh).astype(BF16)
        b1a = mk(1, s1b.at[pl.ds((1 - x) * mq, mq), :],
                 c1b.at[pl.ds((1 - x) * mq, mq), :], py)
        b1a.start()

        out_ref[pl.ds(qa, mq), cols_a] = qdot(qa, 0)
        a1a.wait_recv()
        out_ref[pl.ds(qa, mq), cols_a] += c1a[pl.ds((1 - y) * mq, mq), :].astype(F32)
        s2a[...] = out_ref[pl.ds(qa, mq), cols_a].astype(BF16)
        a2 = mk(2, s2a, c2a, py)
        a2.start()

        out_ref[pl.ds(qb, mq), cols_b] = qdot(qb, dh)
        b1a.wait_recv()
        out_ref[pl.ds(qb, mq), cols_b] += c1b[pl.ds((1 - x) * mq, mq), :].astype(F32)
        s2b[...] = out_ref[pl.ds(qb, mq), cols_b].astype(BF16)
        b2 = mk(3, s2b, c2b, px)
        b2.start()

        s1a[pl.ds(y * mq, mq), :] = qdot((1 - x) * mh + y * mq, 0).astype(BF16)
        a1b = mk(4, s1a.at[pl.ds(y * mq, mq), :],
                 c1a.at[pl.ds(y * mq, mq), :], px)
        a1b.start()
        s1b[pl.ds(x * mq, mq), :] = qdot((1 - y) * mh + x * mq, dh).astype(BF16)
        b1b = mk(5, s1b.at[pl.ds(x * mq, mq), :],
                 c1b.at[pl.ds(x * mq, mq), :], py)
        b1b.start()

        out_ref[pl.ds(ra, mq), cols_a] = qdot(ra, 0)
        a1b.wait_recv()
        out_ref[pl.ds(ra, mq), cols_a] += c1a[pl.ds(y * mq, mq), :].astype(F32)
        a2.wait_recv()
        out_ref[pl.ds(ra, mq), cols_a] += c2a[...].astype(F32)
        s3a[...] = out_ref[pl.ds(ra, mq), cols_a].astype(BF16)
        a3 = mk(6, s3a, r3a, py)
        a3.start()
        a4i = mk(7, s3a, r4a.at[pl.ds(y * mq, mq), :], px)
        a4i.start()

        out_ref[pl.ds(rb, mq), cols_b] = qdot(rb, dh)
        b1b.wait_recv()
        out_ref[pl.ds(rb, mq), cols_b] += c1b[pl.ds(x * mq, mq), :].astype(F32)
        b2.wait_recv()
        out_ref[pl.ds(rb, mq), cols_b] += c2b[...].astype(F32)
        s3b[...] = out_ref[pl.ds(rb, mq), cols_b].astype(BF16)
        b3 = mk(8, s3b, r3b, px)
        b3.start()
        b4i = mk(9, s3b, r4b.at[pl.ds(x * mq, mq), :], py)
        b4i.start()

        a3.wait_recv()
        a4f = mk(10, r3a, r4a.at[pl.ds((1 - y) * mq, mq), :], px)
        a4f.start()
        out_ref[pl.ds(qa, mq), cols_a] = r3a[...].astype(F32)

        b3.wait_recv()
        b4f = mk(11, r3b, r4b.at[pl.ds((1 - x) * mq, mq), :], py)
        b4f.start()
        out_ref[pl.ds(qb, mq), cols_b] = r3b[...].astype(F32)

        a4i.wait_recv()
        a4f.wait_recv()
        out_ref[pl.ds((1 - x) * mh, mh), cols_a] = r4a[...].astype(F32)
        b4i.wait_recv()
        b4f.wait_recv()
        out_ref[pl.ds((1 - y) * mh, mh), cols_b] = r4b[...].astype(F32)

        for r in [a1a, b1a, a2, b2, a1b, b1b, a3, a4i, b3, b4i, a4f, b4f]:
            r.wait_send()

    return pl.pallas_call(
        body,
        out_shape=jax.ShapeDtypeStruct((m, d), F32),
        in_specs=[
            pl.BlockSpec(memory_space=pltpu.VMEM),
            pl.BlockSpec(memory_space=pltpu.VMEM),
        ],
        out_specs=pl.BlockSpec(memory_space=pltpu.VMEM),
        scratch_shapes=[
            pltpu.VMEM((mh, dh), BF16),
            pltpu.VMEM((mh, dh), BF16),
            pltpu.VMEM((mh, dh), BF16),
            pltpu.VMEM((mh, dh), BF16),
            pltpu.VMEM((mq, dh), BF16),
            pltpu.VMEM((mq, dh), BF16),
            pltpu.VMEM((mq, dh), BF16),
            pltpu.VMEM((mq, dh), BF16),
            pltpu.VMEM((mq, dh), BF16),
            pltpu.VMEM((mq, dh), BF16),
            pltpu.VMEM((mq, dh), BF16),
            pltpu.VMEM((mq, dh), BF16),
            pltpu.VMEM((mh, dh), BF16),
            pltpu.VMEM((mh, dh), BF16),
            pltpu.SemaphoreType.DMA((12,)),
            pltpu.SemaphoreType.DMA((12,)),
        ],
        compiler_params=pltpu.CompilerParams(
            collective_id=0, vmem_limit_bytes=100 * 1024 * 1024,
        ),
    )(dy, W)
